# Initial kernel scaffold; baseline (speedup 1.0000x reference)
import jax
import jax.numpy as jnp
from jax.experimental import pallas as pl


def kernel(drug_feat, cell_feat, hyperedge_index, drug_lin_w, drug_lin_b, cell_lin_w, cell_lin_b, linV_w, linE_w, biasV, biasE, ln_g, ln_b):
    raise NotImplementedError("write your pallas kernel here")


# speedup vs baseline: 25.5500x; 25.5500x over previous
import functools

import jax
import jax.numpy as jnp
from jax import lax
from jax.experimental import pallas as pl
from jax.experimental.pallas import tpu as pltpu
from jax.experimental.pallas import tpu_sc as plsc

ALPHA = 0.1
DIM = 128
N_EDGE = 2000
N_ACC = 2048
NNZ = 320000
CHUNK = 128
NUM_CHUNKS = NNZ // CHUNK
NC = 2
NS = 16
NW = NC * NS
FULL_ITERS = NUM_CHUNKS // NW
TAIL = NUM_CHUNKS - FULL_ITERS * NW
ROWS_PER_SUB = N_ACC // NS


def _sc_segment_sum(cell_feat, node_idx, edge_idx, zacc, zcnt, ones_init):
    mesh = plsc.VectorSubcoreMesh(core_axis_name="c", subcore_axis_name="s")

    @functools.partial(
        pl.kernel,
        out_type=(
            jax.ShapeDtypeStruct((NC, N_ACC, DIM), jnp.float32),
            jax.ShapeDtypeStruct((NC, N_ACC, 16), jnp.float32),
        ),
        mesh=mesh,
        scratch_types=[
            pltpu.VMEM((1, CHUNK), jnp.int32),
            pltpu.VMEM((1, CHUNK), jnp.int32),
            pltpu.VMEM((CHUNK, DIM), jnp.float32),
            pltpu.VMEM((CHUNK, 16), jnp.float32),
            pltpu.VMEM_SHARED((N_ACC, DIM), jnp.float32),
            pltpu.VMEM_SHARED((N_ACC, 16), jnp.float32),
            pltpu.SemaphoreType.DMA,
        ],
    )
    def sc_kernel(cell_hbm, nidx_hbm, eidx_hbm, zacc_hbm, zcnt_hbm, ones_hbm,
                  acc_out, cnt_out, nidx_v, eidx_v, rows_v, ones_v,
                  acc_sh, cnt_sh, sem):
        c = lax.axis_index("c")
        s = lax.axis_index("s")
        wid = c * NS + s
        row0 = s * ROWS_PER_SUB

        pltpu.sync_copy(ones_hbm, ones_v)
        pltpu.sync_copy(zacc_hbm.at[pl.ds(row0, ROWS_PER_SUB)],
                        acc_sh.at[pl.ds(row0, ROWS_PER_SUB)])
        pltpu.sync_copy(zcnt_hbm.at[pl.ds(row0, ROWS_PER_SUB)],
                        cnt_sh.at[pl.ds(row0, ROWS_PER_SUB)])
        plsc.subcore_barrier()

        def do_chunk(chunk_id):
            off = chunk_id * CHUNK
            pltpu.sync_copy(nidx_hbm.at[pl.ds(off, CHUNK)], nidx_v.at[0])
            pltpu.sync_copy(eidx_hbm.at[pl.ds(off, CHUNK)], eidx_v.at[0])
            pltpu.async_copy(cell_hbm.at[nidx_v.at[0]], rows_v, sem).wait()
            pltpu.sync_copy(rows_v, acc_sh.at[eidx_v.at[0]], add=True)
            pltpu.sync_copy(ones_v, cnt_sh.at[eidx_v.at[0]], add=True)

        @pl.loop(0, FULL_ITERS)
        def _(t):
            do_chunk(wid + NW * t)

        @pl.when(wid < TAIL)
        def _():
            do_chunk(wid + NW * FULL_ITERS)

        plsc.subcore_barrier()
        pltpu.sync_copy(acc_sh.at[pl.ds(row0, ROWS_PER_SUB)],
                        acc_out.at[c, pl.ds(row0, ROWS_PER_SUB)])
        pltpu.sync_copy(cnt_sh.at[pl.ds(row0, ROWS_PER_SUB)],
                        cnt_out.at[c, pl.ds(row0, ROWS_PER_SUB)])

    return sc_kernel(cell_feat, node_idx, edge_idx, zacc, zcnt, ones_init)


def _tc_dense_body(acc_ref, cnt_ref, df_ref, wd_ref, bd_ref, wc_ref, bc_ref,
                   we_ref, be_ref, g_ref, b_ref, out_ref):
    T0 = acc_ref[0, :N_EDGE, :] + acc_ref[1, :N_EDGE, :]
    cnt = (cnt_ref[0, :N_EDGE, :] + cnt_ref[1, :N_EDGE, :]).sum(
        axis=1, keepdims=True) * (1.0 / 16.0)
    Bg = jnp.where(cnt > 0, lax.rsqrt(cnt), 0.0)
    sq = jnp.sqrt(cnt)

    def matT(x, w):
        return lax.dot_general(x, w, (((1,), (1,)), ((), ())),
                               preferred_element_type=jnp.float32)

    S = matT(Bg * T0, wc_ref[...]) + sq * bc_ref[...]
    feat = matT(df_ref[...], wd_ref[...]) + bd_ref[...]
    for i in range(3):
        h = matT(S, we_ref[i]) + be_ref[i] + ALPHA * feat
        h = h * jax.nn.sigmoid(h)
        m = jnp.mean(h, axis=1, keepdims=True)
        v = jnp.mean((h - m) ** 2, axis=1, keepdims=True)
        feat = (h - m) * lax.rsqrt(v + 1e-5) * g_ref[...] + b_ref[...]
    out_ref[...] = feat


def kernel(drug_feat, cell_feat, hyperedge_index, drug_lin_w, drug_lin_b,
           cell_lin_w, cell_lin_b, linV_w, linE_w, biasV, biasE, ln_g, ln_b):
    node_idx = hyperedge_index[0]
    edge_idx = hyperedge_index[1]
    zacc = jnp.zeros((N_ACC, DIM), jnp.float32)
    zcnt = jnp.zeros((N_ACC, 16), jnp.float32)
    ones_init = jnp.ones((CHUNK, 16), jnp.float32)

    acc, cnt = _sc_segment_sum(cell_feat, node_idx, edge_idx, zacc, zcnt,
                               ones_init)

    out = pl.pallas_call(
        _tc_dense_body,
        out_shape=jax.ShapeDtypeStruct((N_EDGE, DIM), jnp.float32),
    )(acc, cnt, drug_feat,
      drug_lin_w, drug_lin_b.reshape(1, DIM),
      cell_lin_w, cell_lin_b.reshape(1, DIM),
      linE_w, biasE.reshape(3, 1, DIM),
      ln_g.reshape(1, DIM), ln_b.reshape(1, DIM))
    return out


# speedup vs baseline: 36.2534x; 1.4189x over previous
import functools

import jax
import jax.numpy as jnp
from jax import lax
from jax.experimental import pallas as pl
from jax.experimental.pallas import tpu as pltpu
from jax.experimental.pallas import tpu_sc as plsc

ALPHA = 0.1
DIM = 128
N_EDGE = 2000
N_ACC = 2048
NNZ = 320000
CHUNK = 128
NUM_CHUNKS = NNZ // CHUNK
NC = 2
NS = 16
NW = NC * NS
FULL_ITERS = NUM_CHUNKS // NW
TAIL = NUM_CHUNKS - FULL_ITERS * NW
ROWS_PER_SUB = N_ACC // NS
G = 4
BATCH_ITERS = FULL_ITERS // G
G_REM = FULL_ITERS - BATCH_ITERS * G


def _sc_segment_sum(cell_feat, node_idx, edge_idx, zacc, zcnt, ones_init):
    mesh = plsc.VectorSubcoreMesh(core_axis_name="c", subcore_axis_name="s")

    @functools.partial(
        pl.kernel,
        out_type=(
            jax.ShapeDtypeStruct((NC, N_ACC, DIM), jnp.float32),
            jax.ShapeDtypeStruct((NC, N_ACC, 16), jnp.float32),
        ),
        mesh=mesh,
        scratch_types=[
            pltpu.VMEM((G, CHUNK), jnp.int32),
            pltpu.VMEM((G, CHUNK), jnp.int32),
            pltpu.VMEM((G, CHUNK, DIM), jnp.float32),
            pltpu.VMEM((CHUNK, 16), jnp.float32),
            pltpu.VMEM_SHARED((N_ACC, DIM), jnp.float32),
            pltpu.VMEM_SHARED((N_ACC, 16), jnp.float32),
            pltpu.SemaphoreType.DMA((G,)),
            pltpu.SemaphoreType.DMA((G,)),
            pltpu.SemaphoreType.DMA((G,)),
            pltpu.SemaphoreType.DMA,
        ],
    )
    def sc_kernel(cell_hbm, nidx_hbm, eidx_hbm, zacc_hbm, zcnt_hbm, ones_hbm,
                  acc_out, cnt_out, nidx_v, eidx_v, rows_v, ones_v,
                  acc_sh, cnt_sh, sem_n, sem_e, sem_g, sem_s):
        c = lax.axis_index("c")
        s = lax.axis_index("s")
        wid = c * NS + s
        row0 = s * ROWS_PER_SUB

        pltpu.sync_copy(ones_hbm, ones_v)
        pltpu.sync_copy(zacc_hbm.at[pl.ds(row0, ROWS_PER_SUB)],
                        acc_sh.at[pl.ds(row0, ROWS_PER_SUB)])
        pltpu.sync_copy(zcnt_hbm.at[pl.ds(row0, ROWS_PER_SUB)],
                        cnt_sh.at[pl.ds(row0, ROWS_PER_SUB)])
        plsc.subcore_barrier()

        def do_batch(t0, count):
            hi = []
            for g in range(count):
                off = (wid + NW * (t0 + g)) * CHUNK
                hi.append(pltpu.async_copy(nidx_hbm.at[pl.ds(off, CHUNK)],
                                           nidx_v.at[g], sem_n.at[g]))
                hi.append(pltpu.async_copy(eidx_hbm.at[pl.ds(off, CHUNK)],
                                           eidx_v.at[g], sem_e.at[g]))
            for h in hi:
                h.wait()
            hg = []
            for g in range(count):
                hg.append(pltpu.async_copy(cell_hbm.at[nidx_v.at[g]],
                                           rows_v.at[g], sem_g.at[g]))
            for h in hg:
                h.wait()
            for g in range(count):
                pltpu.sync_copy(rows_v.at[g], acc_sh.at[eidx_v.at[g]],
                                add=True)
                pltpu.sync_copy(ones_v, cnt_sh.at[eidx_v.at[g]], add=True)

        @pl.loop(0, BATCH_ITERS)
        def _(t):
            do_batch(t * G, G)

        if G_REM:
            do_batch(BATCH_ITERS * G, G_REM)

        @pl.when(wid < TAIL)
        def _():
            off = (wid + NW * FULL_ITERS) * CHUNK
            pltpu.sync_copy(nidx_hbm.at[pl.ds(off, CHUNK)], nidx_v.at[0])
            pltpu.sync_copy(eidx_hbm.at[pl.ds(off, CHUNK)], eidx_v.at[0])
            pltpu.async_copy(cell_hbm.at[nidx_v.at[0]], rows_v.at[0],
                             sem_g.at[0]).wait()
            pltpu.sync_copy(rows_v.at[0], acc_sh.at[eidx_v.at[0]], add=True)
            pltpu.sync_copy(ones_v, cnt_sh.at[eidx_v.at[0]], add=True)

        plsc.subcore_barrier()
        pltpu.sync_copy(acc_sh.at[pl.ds(row0, ROWS_PER_SUB)],
                        acc_out.at[c, pl.ds(row0, ROWS_PER_SUB)])
        pltpu.sync_copy(cnt_sh.at[pl.ds(row0, ROWS_PER_SUB)],
                        cnt_out.at[c, pl.ds(row0, ROWS_PER_SUB)])

    return sc_kernel(cell_feat, node_idx, edge_idx, zacc, zcnt, ones_init)


def _tc_dense_body(acc_ref, cnt_ref, df_ref, wd_ref, bd_ref, wc_ref, bc_ref,
                   we_ref, be_ref, g_ref, b_ref, out_ref):
    T0 = acc_ref[0, :N_EDGE, :] + acc_ref[1, :N_EDGE, :]
    cnt = (cnt_ref[0, :N_EDGE, :] + cnt_ref[1, :N_EDGE, :]).sum(
        axis=1, keepdims=True) * (1.0 / 16.0)
    Bg = jnp.where(cnt > 0, lax.rsqrt(cnt), 0.0)
    sq = jnp.sqrt(cnt)

    def matT(x, w):
        return lax.dot_general(x, w, (((1,), (1,)), ((), ())),
                               preferred_element_type=jnp.float32)

    S = matT(Bg * T0, wc_ref[...]) + sq * bc_ref[...]
    feat = matT(df_ref[...], wd_ref[...]) + bd_ref[...]
    for i in range(3):
        h = matT(S, we_ref[i]) + be_ref[i] + ALPHA * feat
        h = h * jax.nn.sigmoid(h)
        m = jnp.mean(h, axis=1, keepdims=True)
        v = jnp.mean((h - m) ** 2, axis=1, keepdims=True)
        feat = (h - m) * lax.rsqrt(v + 1e-5) * g_ref[...] + b_ref[...]
    out_ref[...] = feat


def kernel(drug_feat, cell_feat, hyperedge_index, drug_lin_w, drug_lin_b,
           cell_lin_w, cell_lin_b, linV_w, linE_w, biasV, biasE, ln_g, ln_b):
    node_idx = hyperedge_index[0]
    edge_idx = hyperedge_index[1]
    zacc = jnp.zeros((N_ACC, DIM), jnp.float32)
    zcnt = jnp.zeros((N_ACC, 16), jnp.float32)
    ones_init = jnp.ones((CHUNK, 16), jnp.float32)

    acc, cnt = _sc_segment_sum(cell_feat, node_idx, edge_idx, zacc, zcnt,
                               ones_init)

    out = pl.pallas_call(
        _tc_dense_body,
        out_shape=jax.ShapeDtypeStruct((N_EDGE, DIM), jnp.float32),
    )(acc, cnt, drug_feat,
      drug_lin_w, drug_lin_b.reshape(1, DIM),
      cell_lin_w, cell_lin_b.reshape(1, DIM),
      linE_w, biasE.reshape(3, 1, DIM),
      ln_g.reshape(1, DIM), ln_b.reshape(1, DIM))
    return out
